# baseline (speedup 1.0000x reference)
import functools
import jax
import jax.numpy as jnp
from jax import lax
from jax.experimental import pallas as pl
from jax.experimental.pallas import tpu as pltpu, tpu_sc as plsc

_NUM_CODEBOOKS = 8
_VOCAB = 100000
_D = 64
_N = 1024 * 50 * 8
_NC, _NS, _L = 2, 16, 16
_NW = _NC * _NS
_N_PER_W = _N // _NW
_CHUNK = 128
_NCHUNK = _N_PER_W // _CHUNK


def _make_gather():
    mesh = plsc.VectorSubcoreMesh(core_axis_name="c", subcore_axis_name="s")

    @functools.partial(
        pl.kernel,
        mesh=mesh,
        out_type=jax.ShapeDtypeStruct((_N, _D), jnp.float32),
        compiler_params=pltpu.CompilerParams(use_tc_tiling_on_sc=False),
        scratch_types=[
            pltpu.VMEM((_NCHUNK, _CHUNK), jnp.int32),
            pltpu.VMEM((_CHUNK, _D), jnp.float32),
            pltpu.SemaphoreType.DMA,
        ],
    )
    def gather_kernel(tok_hbm, table_hbm, out_hbm, idx_v, rows_v, sem):
        wid = lax.axis_index("s") * _NC + lax.axis_index("c")
        base = wid * _N_PER_W

        pltpu.sync_copy(tok_hbm.at[wid], idx_v)

        offs = (lax.iota(jnp.int32, 16) % _NUM_CODEBOOKS) * _VOCAB

        def add_offsets(j, _):
            for l in range(_CHUNK // _L):
                sl = pl.ds(l * _L, _L)
                idx_v[j, sl] = idx_v[j, sl] + offs
            return 0

        lax.fori_loop(0, _NCHUNK, add_offsets, 0)

        def do_chunk(j, _):
            pltpu.async_copy(table_hbm.at[idx_v.at[j]], rows_v, sem).wait()
            pltpu.sync_copy(rows_v, out_hbm.at[pl.ds(base + j * _CHUNK, _CHUNK)])
            return 0

        lax.fori_loop(0, _NCHUNK, do_chunk, 0)

    return gather_kernel


_gather = _make_gather()


@jax.jit
def kernel(in_tokens, embedding_weight):
    tok_flat = in_tokens.reshape(_NW, _NCHUNK, _CHUNK)
    out = _gather(tok_flat, embedding_weight)
    return out.reshape(in_tokens.shape + (_D,))


# speedup vs baseline: 1.0918x; 1.0918x over previous
import functools
import jax
import jax.numpy as jnp
from jax import lax
from jax.experimental import pallas as pl
from jax.experimental.pallas import tpu as pltpu, tpu_sc as plsc

_NUM_CODEBOOKS = 8
_VOCAB = 100000
_D = 64
_N = 1024 * 50 * 8
_NC, _NS, _L = 2, 16, 16
_NW = _NC * _NS
_N_PER_W = _N // _NW
_CHUNK = 128
_NCHUNK = _N_PER_W // _CHUNK
_NBUF = 4
_NGRP = _NCHUNK // _NBUF


def _make_gather():
    mesh = plsc.VectorSubcoreMesh(core_axis_name="c", subcore_axis_name="s")

    @functools.partial(
        pl.kernel,
        mesh=mesh,
        out_type=jax.ShapeDtypeStruct((_N, _D), jnp.float32),
        compiler_params=pltpu.CompilerParams(use_tc_tiling_on_sc=False),
        scratch_types=[
            pltpu.VMEM((_NCHUNK, _CHUNK), jnp.int32),
            pltpu.VMEM((2, _NBUF, _CHUNK, _D), jnp.float32),
            pltpu.SemaphoreType.DMA,
            pltpu.SemaphoreType.DMA,
        ],
    )
    def gather_kernel(tok_hbm, table_hbm, out_hbm, idx_v, rows_v, sem_g, sem_s):
        wid = lax.axis_index("s") * _NC + lax.axis_index("c")
        base = wid * _N_PER_W

        pltpu.sync_copy(tok_hbm.at[wid], idx_v)

        offs = (lax.iota(jnp.int32, 16) % _NUM_CODEBOOKS) * _VOCAB

        def add_offsets(j, _):
            for l in range(_CHUNK // _L):
                sl = pl.ds(l * _L, _L)
                idx_v[j, sl] = idx_v[j, sl] + offs
            return 0

        lax.fori_loop(0, _NCHUNK, add_offsets, 0)

        def gather_copy(j, p, b):
            return pltpu.make_async_copy(
                table_hbm.at[idx_v.at[j]], rows_v.at[p, b], sem_g)

        def store_copy(j, p, b):
            return pltpu.make_async_copy(
                rows_v.at[p, b], out_hbm.at[pl.ds(base + j * _CHUNK, _CHUNK)],
                sem_s)

        for b in range(_NBUF):
            gather_copy(b, 0, b).start()

        def group(g, _):
            p = g % 2

            @pl.when(g >= 1)
            def _():
                for b in range(_NBUF):
                    store_copy((g - 1) * _NBUF + b, 1 - p, b).wait()

            @pl.when(g + 1 < _NGRP)
            def _():
                for b in range(_NBUF):
                    gather_copy((g + 1) * _NBUF + b, 1 - p, b).start()

            for b in range(_NBUF):
                j = g * _NBUF + b
                gather_copy(j, p, b).wait()
                store_copy(j, p, b).start()
            return 0

        lax.fori_loop(0, _NGRP, group, 0)

        for b in range(_NBUF):
            store_copy((_NGRP - 1) * _NBUF + b, (_NGRP - 1) % 2, b).wait()

    return gather_kernel


_gather = _make_gather()


@jax.jit
def kernel(in_tokens, embedding_weight):
    tok_flat = in_tokens.reshape(_NW, _NCHUNK, _CHUNK)
    out = _gather(tok_flat, embedding_weight)
    return out.reshape(in_tokens.shape + (_D,))
